# baseline (speedup 1.0000x reference)
import jax
import jax.numpy as jnp
from jax import lax
from jax.experimental import pallas as pl
from jax.experimental.pallas import tpu as pltpu
from jax.experimental.pallas import tpu_sc as plsc

B = 200
NB = 50
DM = 128
NS = 11
NSP = 16

NC = 2
NSC = 16
NW = NC * NSC
CH = 128

RB = 400
GPB = RB // NB


def _tc_enc_body(sem_ref, geo_ref, wg_ref, tsem_ref, tpos_ref, b0_ref, w_ref,
                 a_ref, bm_ref, g_ref):
    sem = sem_ref[...]
    onehot = (sem == lax.broadcasted_iota(jnp.int32, (1, NSP), 1)
              ).astype(jnp.float32)
    pos = jnp.tile(tpos_ref[...], (GPB, 1))
    h = (geo_ref[...] @ wg_ref[...] + onehot @ tsem_ref[...]
         + pos + b0_ref[...])
    h = jnp.maximum(h, 0.0)
    w = w_ref[...]
    a_ref[...] = h @ w[:DM]
    bm_ref[...] = h @ w[DM:]
    for k in range(GPB):
        g_ref[pl.ds(k, 1), :] = jnp.max(h[k * NB:(k + 1) * NB, :], axis=0,
                                        keepdims=True)


def _tc_mid_body(a_ref, s0_ref, s1_ref, cnt_ref, bl_ref, w_ref,
                 a2_ref, b2_ref, g_ref):
    cnt = jnp.sum(cnt_ref[...], axis=1, keepdims=True)
    scale = 1.0 / jnp.maximum(cnt, 1.0)
    ind = (cnt > 0.0).astype(jnp.float32)
    s = s0_ref[...] + s1_ref[...]
    h = jnp.maximum((a_ref[...] + bl_ref[...]) * ind + s * scale, 0.0)
    w = w_ref[...]
    a2_ref[...] = h @ w[:DM]
    b2_ref[...] = h @ w[DM:]
    for k in range(GPB):
        g_ref[pl.ds(k, 1), :] = jnp.max(h[k * NB:(k + 1) * NB, :], axis=0,
                                        keepdims=True)


def _tc_last_body(a_ref, s0_ref, s1_ref, cnt_ref, bl_ref, g_ref):
    cnt = jnp.sum(cnt_ref[...], axis=1, keepdims=True)
    scale = 1.0 / jnp.maximum(cnt, 1.0)
    ind = (cnt > 0.0).astype(jnp.float32)
    s = s0_ref[...] + s1_ref[...]
    h = jnp.maximum((a_ref[...] + bl_ref[...]) * ind + s * scale, 0.0)
    for k in range(GPB):
        g_ref[pl.ds(k, 1), :] = jnp.max(h[k * NB:(k + 1) * NB, :], axis=0,
                                        keepdims=True)


def _tc_head_body(g0_ref, g1_ref, g2_ref, g3_ref, wa_ref, ba_ref,
                  wmu_ref, bmu_ref, wv_ref, bv_ref, mu_ref, lv_ref):
    wa = wa_ref[...]
    lat = (g0_ref[...] @ wa[:DM] + g1_ref[...] @ wa[DM:2 * DM]
           + g2_ref[...] @ wa[2 * DM:3 * DM] + g3_ref[...] @ wa[3 * DM:]
           + ba_ref[...])
    mu_ref[...] = lat @ wmu_ref[...] + bmu_ref[...]
    lv_ref[...] = lat @ wv_ref[...] + bv_ref[...]


def _row_spec(i_map=None):
    return pl.BlockSpec((RB, DM), i_map or (lambda i: (i, 0)))


def _full(shape):
    return pl.BlockSpec(shape, lambda i: tuple(0 for _ in shape))


def _make_sc_cnt(ept, cnt_r):
    out_type = jax.ShapeDtypeStruct((NW * cnt_r,), jnp.float32)
    scratch = [
        pltpu.VMEM((ept,), jnp.int32),
        pltpu.VMEM((cnt_r,), jnp.float32),
    ]

    def body(dstp, cnt_out, dst_f, cnt_v):
        cid = lax.axis_index("c")
        sid = lax.axis_index("s")
        wid = sid * NC + cid
        pltpu.sync_copy(dstp.at[pl.ds(wid * ept, ept)], dst_f)
        zero16 = jnp.zeros((16,), jnp.float32)

        @pl.loop(0, cnt_r // 16)
        def _(i):
            cnt_v[pl.ds(pl.multiple_of(i * 16, 16), 16)] = zero16

        ones16 = jnp.ones((16,), jnp.float32)

        @pl.loop(0, ept // 16)
        def _(i):
            dk = dst_f[pl.ds(pl.multiple_of(i * 16, 16), 16)]
            plsc.addupdate_scatter(cnt_v, [dk], ones16)

        pltpu.sync_copy(cnt_v, cnt_out.at[pl.ds(wid * cnt_r, cnt_r)])

    mesh = plsc.VectorSubcoreMesh(core_axis_name="c", subcore_axis_name="s")
    return pl.kernel(body, out_type=out_type, mesh=mesh,
                     scratch_types=tuple(scratch),
                     compiler_params=pltpu.CompilerParams(
                         needs_layout_passes=False))


def _make_sc_scatter(ept, cpt, acc_r):
    tpt = acc_r // NSC
    zr = 40
    out_type = [jax.ShapeDtypeStruct((acc_r, DM), jnp.float32),
                jax.ShapeDtypeStruct((acc_r, DM), jnp.float32)]
    scratch = [
        pltpu.VMEM((2, CH), jnp.int32),
        pltpu.VMEM((cpt, CH), jnp.int32),
        pltpu.VMEM((CH, DM), jnp.float32),
        pltpu.VMEM((CH, DM), jnp.float32),
        pltpu.VMEM((zr, DM), jnp.float32),
        pltpu.VMEM_SHARED((acc_r, DM), jnp.float32),
        pltpu.SemaphoreType.DMA,
        pltpu.SemaphoreType.DMA,
        pltpu.SemaphoreType.DMA,
        pltpu.SemaphoreType.DMA,
    ]

    def body(bm, srcp, dst2d, zrows, s0_out, s1_out,
             sv, dst_v, gb0, gb1, zbuf, acc, isem, gsem, gsem2, ssem):
        cid = lax.axis_index("c")
        sid = lax.axis_index("s")
        wid = sid * NC + cid
        ebase = wid * ept

        pltpu.sync_copy(zrows, zbuf)
        nfull = tpt // zr
        for j in range(nfull):
            pltpu.sync_copy(zbuf, acc.at[pl.ds(sid * tpt + j * zr, zr)])
        rem = tpt - nfull * zr
        if rem:
            pltpu.sync_copy(zbuf.at[pl.ds(0, rem)],
                            acc.at[pl.ds(sid * tpt + nfull * zr, rem)])

        pltpu.sync_copy(dst2d.at[pl.ds(wid * cpt, cpt)], dst_v)
        plsc.subcore_barrier()

        svs = (sv.at[0], sv.at[1])
        gbs = (gb0, gb1)

        def idx_copy(c, b):
            off = pl.multiple_of(ebase + c * CH, CH)
            return pltpu.make_async_copy(srcp.at[pl.ds(off, CH)], svs[b],
                                         isem)

        H = CH // 2

        def gather_descs(b):
            lo = pltpu.make_async_copy(bm.at[svs[b].at[pl.ds(0, H)]],
                                       gbs[b].at[pl.ds(0, H)], gsem)
            hi = pltpu.make_async_copy(bm.at[svs[b].at[pl.ds(H, H)]],
                                       gbs[b].at[pl.ds(H, H)], gsem2)
            return lo, hi

        def gather_start(b):
            lo, hi = gather_descs(b)
            lo.start()
            hi.start()

        def gather_wait(b):
            lo, hi = gather_descs(b)
            lo.wait()
            hi.wait()

        def scat_wait(c, b):
            pltpu.make_async_copy(gbs[b], acc.at[dst_v.at[c]], ssem).wait()

        idx_copy(0, 0).start()
        idx_copy(0, 0).wait()
        gather_start(0)
        idx_copy(1, 1).start()

        @pl.loop(0, cpt // 2)
        def _(it):
            c0 = it * 2
            for b in range(2):
                c = c0 + b

                @pl.when(c + 1 < cpt)
                def _():
                    idx_copy(c + 1, 1 - b).wait()

                    @pl.when(c >= 1)
                    def _():
                        scat_wait(c - 1, 1 - b)
                    gather_start(1 - b)

                gather_wait(b)
                pltpu.async_copy(gbs[b], acc.at[dst_v.at[c]], ssem,
                                 add=True)

                @pl.when(c + 2 < cpt)
                def _():
                    idx_copy(c + 2, b).start()

        scat_wait(cpt - 2, 0)
        scat_wait(cpt - 1, 1)

        plsc.subcore_barrier()

        row0 = sid * tpt

        @pl.when(cid == 0)
        def _():
            pltpu.sync_copy(acc.at[pl.ds(row0, tpt)],
                            s0_out.at[pl.ds(row0, tpt)])

        @pl.when(cid == 1)
        def _():
            pltpu.sync_copy(acc.at[pl.ds(row0, tpt)],
                            s1_out.at[pl.ds(row0, tpt)])

    mesh = plsc.VectorSubcoreMesh(core_axis_name="c", subcore_axis_name="s")
    return pl.kernel(body, out_type=tuple(out_type), mesh=mesh,
                     scratch_types=tuple(scratch),
                     compiler_params=pltpu.CompilerParams(
                         needs_layout_passes=False))


def kernel(geometry, semantic, edge_index, batch, ptr, Wg, bg, emb, Wlot,
           blot, W1, b1, W2, b2, W3, b3, Wagg, bagg, Wmu, bmu, Wvar, bvar):
    n = geometry.shape[0]
    e = edge_index.shape[1]
    g_blocks = n // RB
    dump = n + 8
    acc_r = -(-(dump + 1) // 128) * 128
    cpt = -(-e // (NW * CH))
    cpt += cpt % 2
    ept = cpt * CH
    epad = ept * NW
    cnt_r = acc_r

    f32 = jnp.float32
    geo8 = jnp.pad(geometry, ((0, 0), (0, 3)))
    wg2 = jnp.pad(Wg @ Wlot[:DM], ((0, 3), (0, 0)))
    tsem = jnp.pad(emb @ Wlot[DM:2 * DM], ((0, NSP - NS), (0, 0)))
    tpos = Wlot[2 * DM:]
    b0 = (bg @ Wlot[:DM] + blot)[None, :]
    sem2d = semantic[:, None]

    src = edge_index[0]
    dst = edge_index[1]
    padn = epad - e
    if padn and e % NW == 0:
        ppt = padn // NW
        spread = (n + jnp.arange(ppt, dtype=jnp.int32)
                  % jnp.int32(acc_r - n))
        srcp = jnp.concatenate(
            [src.reshape(NW, e // NW),
             jnp.zeros((NW, ppt), jnp.int32)], axis=1).reshape(-1)
        dstp = jnp.concatenate(
            [dst.reshape(NW, e // NW),
             jnp.tile(spread, (NW, 1))], axis=1).reshape(-1)
    else:
        srcp = jnp.concatenate([src, jnp.zeros((padn,), jnp.int32)])
        dstp = jnp.concatenate([dst, jnp.full((padn,), dump, jnp.int32)])
    dst2d = dstp.reshape(NW * cpt, CH)
    zrows = jnp.zeros((40, DM), f32)

    grid = (g_blocks,)
    enc = pl.pallas_call(
        _tc_enc_body,
        grid=grid,
        in_specs=[
            pl.BlockSpec((RB, 1), lambda i: (i, 0)),
            pl.BlockSpec((RB, 8), lambda i: (i, 0)),
            _full((8, DM)), _full((NSP, DM)), _full((NB, DM)),
            _full((1, DM)), _full((2 * DM, DM)),
        ],
        out_specs=[_row_spec(), _row_spec(),
                   pl.BlockSpec((GPB, DM), lambda i: (i, 0))],
        out_shape=[jax.ShapeDtypeStruct((n, DM), f32),
                   jax.ShapeDtypeStruct((n, DM), f32),
                   jax.ShapeDtypeStruct((B, DM), f32)],
    )
    a1, bm1, g0 = enc(sem2d, geo8, wg2, tsem, tpos, b0, W1)

    cntp = _make_sc_cnt(ept, cnt_r)(dstp)
    cnt_t = cntp.reshape(NW, cnt_r).T
    sc_scatter = _make_sc_scatter(ept, cpt, acc_r)

    def mid(a, s0, s1, bl, wn):
        call = pl.pallas_call(
            _tc_mid_body,
            grid=grid,
            in_specs=[
                _row_spec(), _row_spec(), _row_spec(),
                pl.BlockSpec((RB, NW), lambda i: (i, 0)),
                _full((1, DM)), _full((2 * DM, DM)),
            ],
            out_specs=[_row_spec(), _row_spec(),
                       pl.BlockSpec((GPB, DM), lambda i: (i, 0))],
            out_shape=[jax.ShapeDtypeStruct((n, DM), f32),
                       jax.ShapeDtypeStruct((n, DM), f32),
                       jax.ShapeDtypeStruct((B, DM), f32)],
        )
        return call(a, s0, s1, cnt_t, bl[None, :], wn)

    s1a, s1b = sc_scatter(bm1, srcp, dst2d, zrows)
    a2, bm2, g1 = mid(a1, s1a, s1b, b1, W2)
    s2a, s2b = sc_scatter(bm2, srcp, dst2d, zrows)
    a3, bm3, g2 = mid(a2, s2a, s2b, b2, W3)
    s3a, s3b = sc_scatter(bm3, srcp, dst2d, zrows)

    last = pl.pallas_call(
        _tc_last_body,
        grid=grid,
        in_specs=[
            _row_spec(), _row_spec(), _row_spec(),
            pl.BlockSpec((RB, NW), lambda i: (i, 0)),
            _full((1, DM)),
        ],
        out_specs=[pl.BlockSpec((GPB, DM), lambda i: (i, 0))],
        out_shape=[jax.ShapeDtypeStruct((B, DM), f32)],
    )
    (g3,) = last(a3, s3a, s3b, cnt_t, b3[None, :])

    head = pl.pallas_call(
        _tc_head_body,
        grid=(1,),
        in_specs=[_full((B, DM))] * 4 + [
            _full((4 * DM, DM)), _full((1, DM)),
            _full((DM, DM)), _full((1, DM)),
            _full((DM, DM)), _full((1, DM)),
        ],
        out_specs=[_full((B, DM)), _full((B, DM))],
        out_shape=[jax.ShapeDtypeStruct((B, DM), f32),
                   jax.ShapeDtypeStruct((B, DM), f32)],
    )
    mu, log_var = head(g0, g1, g2, g3, Wagg, bagg[None, :], Wmu,
                       bmu[None, :], Wvar, bvar[None, :])
    return (mu, log_var)


# speedup vs baseline: 1.0100x; 1.0100x over previous
import jax
import jax.numpy as jnp
from jax import lax
from jax.experimental import pallas as pl
from jax.experimental.pallas import tpu as pltpu
from jax.experimental.pallas import tpu_sc as plsc

B = 200
NB = 50
DM = 128
NS = 11
NSP = 16

NC = 2
NSC = 16
NW = NC * NSC
CH = 128

RB = 400
GPB = RB // NB


def _tc_enc_body(sem_ref, geo_ref, wg_ref, tsem_ref, tpos_ref, b0_ref, w_ref,
                 a_ref, bm_ref, g_ref):
    sem = sem_ref[...]
    onehot = (sem == lax.broadcasted_iota(jnp.int32, (1, NSP), 1)
              ).astype(jnp.float32)
    pos = jnp.tile(tpos_ref[...], (GPB, 1))
    h = (geo_ref[...] @ wg_ref[...] + onehot @ tsem_ref[...]
         + pos + b0_ref[...])
    h = jnp.maximum(h, 0.0)
    w = w_ref[...]
    a_ref[...] = h @ w[:DM]
    bm_ref[...] = h @ w[DM:]
    for k in range(GPB):
        g_ref[pl.ds(k, 1), :] = jnp.max(h[k * NB:(k + 1) * NB, :], axis=0,
                                        keepdims=True)


def _tc_mid_body(a_ref, s0_ref, s1_ref, cnt_ref, bl_ref, w_ref,
                 a2_ref, b2_ref, g_ref):
    cnt = jnp.sum(cnt_ref[...], axis=1, keepdims=True)
    scale = 1.0 / jnp.maximum(cnt, 1.0)
    ind = (cnt > 0.0).astype(jnp.float32)
    s = s0_ref[...] + s1_ref[...]
    h = jnp.maximum((a_ref[...] + bl_ref[...]) * ind + s * scale, 0.0)
    w = w_ref[...]
    a2_ref[...] = h @ w[:DM]
    b2_ref[...] = h @ w[DM:]
    for k in range(GPB):
        g_ref[pl.ds(k, 1), :] = jnp.max(h[k * NB:(k + 1) * NB, :], axis=0,
                                        keepdims=True)


def _tc_last_body(a_ref, s0_ref, s1_ref, cnt_ref, bl_ref, g_ref):
    cnt = jnp.sum(cnt_ref[...], axis=1, keepdims=True)
    scale = 1.0 / jnp.maximum(cnt, 1.0)
    ind = (cnt > 0.0).astype(jnp.float32)
    s = s0_ref[...] + s1_ref[...]
    h = jnp.maximum((a_ref[...] + bl_ref[...]) * ind + s * scale, 0.0)
    for k in range(GPB):
        g_ref[pl.ds(k, 1), :] = jnp.max(h[k * NB:(k + 1) * NB, :], axis=0,
                                        keepdims=True)


def _tc_head_body(g0_ref, g1_ref, g2_ref, g3_ref, wa_ref, ba_ref,
                  wmu_ref, bmu_ref, wv_ref, bv_ref, mu_ref, lv_ref):
    wa = wa_ref[...]
    lat = (g0_ref[...] @ wa[:DM] + g1_ref[...] @ wa[DM:2 * DM]
           + g2_ref[...] @ wa[2 * DM:3 * DM] + g3_ref[...] @ wa[3 * DM:]
           + ba_ref[...])
    mu_ref[...] = lat @ wmu_ref[...] + bmu_ref[...]
    lv_ref[...] = lat @ wv_ref[...] + bv_ref[...]


def _row_spec(i_map=None):
    return pl.BlockSpec((RB, DM), i_map or (lambda i: (i, 0)))


def _full(shape):
    return pl.BlockSpec(shape, lambda i: tuple(0 for _ in shape))


def _make_sc_cnt(ept, cnt_r):
    out_type = jax.ShapeDtypeStruct((NW * cnt_r,), jnp.float32)
    scratch = [
        pltpu.VMEM((ept,), jnp.int32),
        pltpu.VMEM((cnt_r,), jnp.float32),
    ]

    def body(dstp, cnt_out, dst_f, cnt_v):
        cid = lax.axis_index("c")
        sid = lax.axis_index("s")
        wid = sid * NC + cid
        pltpu.sync_copy(dstp.at[pl.ds(wid * ept, ept)], dst_f)
        zero16 = jnp.zeros((16,), jnp.float32)

        @pl.loop(0, cnt_r // 16)
        def _(i):
            cnt_v[pl.ds(pl.multiple_of(i * 16, 16), 16)] = zero16

        ones16 = jnp.ones((16,), jnp.float32)

        @pl.loop(0, ept // 16)
        def _(i):
            dk = dst_f[pl.ds(pl.multiple_of(i * 16, 16), 16)]
            plsc.addupdate_scatter(cnt_v, [dk], ones16)

        pltpu.sync_copy(cnt_v, cnt_out.at[pl.ds(wid * cnt_r, cnt_r)])

    mesh = plsc.VectorSubcoreMesh(core_axis_name="c", subcore_axis_name="s")
    return pl.kernel(body, out_type=out_type, mesh=mesh,
                     scratch_types=tuple(scratch),
                     compiler_params=pltpu.CompilerParams(
                         needs_layout_passes=False))


def _make_sc_scatter(ept, cpt, acc_r):
    tpt = acc_r // NSC
    zr = 40
    out_type = [jax.ShapeDtypeStruct((acc_r, DM), jnp.float32),
                jax.ShapeDtypeStruct((acc_r, DM), jnp.float32)]
    scratch = [
        pltpu.VMEM((2, CH), jnp.int32),
        pltpu.VMEM((cpt, CH), jnp.int32),
        pltpu.VMEM((CH, DM), jnp.float32),
        pltpu.VMEM((CH, DM), jnp.float32),
        pltpu.VMEM((zr, DM), jnp.float32),
        pltpu.VMEM_SHARED((acc_r, DM), jnp.float32),
        pltpu.SemaphoreType.DMA,
        pltpu.SemaphoreType.DMA,
        pltpu.SemaphoreType.DMA,
    ]

    def body(bm, srcp, dst2d, zrows, s0_out, s1_out,
             sv, dst_v, gb0, gb1, zbuf, acc, isem, gsem, ssem):
        cid = lax.axis_index("c")
        sid = lax.axis_index("s")
        wid = sid * NC + cid
        ebase = wid * ept

        pltpu.sync_copy(zrows, zbuf)
        nfull = tpt // zr
        for j in range(nfull):
            pltpu.sync_copy(zbuf, acc.at[pl.ds(sid * tpt + j * zr, zr)])
        rem = tpt - nfull * zr
        if rem:
            pltpu.sync_copy(zbuf.at[pl.ds(0, rem)],
                            acc.at[pl.ds(sid * tpt + nfull * zr, rem)])

        pltpu.sync_copy(dst2d.at[pl.ds(wid * cpt, cpt)], dst_v)
        plsc.subcore_barrier()

        svs = (sv.at[0], sv.at[1])
        gbs = (gb0, gb1)

        def idx_copy(c, b):
            off = pl.multiple_of(ebase + c * CH, CH)
            return pltpu.make_async_copy(srcp.at[pl.ds(off, CH)], svs[b],
                                         isem)

        def gather(b):
            return pltpu.make_async_copy(bm.at[svs[b]], gbs[b], gsem)

        def scat_wait(c, b):
            pltpu.make_async_copy(gbs[b], acc.at[dst_v.at[c]], ssem).wait()

        idx_copy(0, 0).start()
        idx_copy(0, 0).wait()
        gather(0).start()
        idx_copy(1, 1).start()

        @pl.loop(0, cpt // 2)
        def _(it):
            c0 = it * 2
            for b in range(2):
                c = c0 + b

                @pl.when(c + 1 < cpt)
                def _():
                    idx_copy(c + 1, 1 - b).wait()

                    @pl.when(c >= 1)
                    def _():
                        scat_wait(c - 1, 1 - b)
                    gather(1 - b).start()

                gather(b).wait()
                pltpu.async_copy(gbs[b], acc.at[dst_v.at[c]], ssem,
                                 add=True)

                @pl.when(c + 2 < cpt)
                def _():
                    idx_copy(c + 2, b).start()

        scat_wait(cpt - 2, 0)
        scat_wait(cpt - 1, 1)

        plsc.subcore_barrier()

        row0 = sid * tpt

        @pl.when(cid == 0)
        def _():
            pltpu.sync_copy(acc.at[pl.ds(row0, tpt)],
                            s0_out.at[pl.ds(row0, tpt)])

        @pl.when(cid == 1)
        def _():
            pltpu.sync_copy(acc.at[pl.ds(row0, tpt)],
                            s1_out.at[pl.ds(row0, tpt)])

    mesh = plsc.VectorSubcoreMesh(core_axis_name="c", subcore_axis_name="s")
    return pl.kernel(body, out_type=tuple(out_type), mesh=mesh,
                     scratch_types=tuple(scratch),
                     compiler_params=pltpu.CompilerParams(
                         needs_layout_passes=False,
                         use_tc_tiling_on_sc=False))


def kernel(geometry, semantic, edge_index, batch, ptr, Wg, bg, emb, Wlot,
           blot, W1, b1, W2, b2, W3, b3, Wagg, bagg, Wmu, bmu, Wvar, bvar):
    n = geometry.shape[0]
    e = edge_index.shape[1]
    g_blocks = n // RB
    dump = n + 8
    acc_r = -(-(dump + 1) // 128) * 128
    cpt = -(-e // (NW * CH))
    cpt += cpt % 2
    ept = cpt * CH
    epad = ept * NW
    cnt_r = acc_r

    f32 = jnp.float32
    geo8 = jnp.pad(geometry, ((0, 0), (0, 3)))
    wg2 = jnp.pad(Wg @ Wlot[:DM], ((0, 3), (0, 0)))
    tsem = jnp.pad(emb @ Wlot[DM:2 * DM], ((0, NSP - NS), (0, 0)))
    tpos = Wlot[2 * DM:]
    b0 = (bg @ Wlot[:DM] + blot)[None, :]
    sem2d = semantic[:, None]

    src = edge_index[0]
    dst = edge_index[1]
    padn = epad - e
    if padn and e % NW == 0:
        ppt = padn // NW
        spread = (n + jnp.arange(ppt, dtype=jnp.int32)
                  % jnp.int32(acc_r - n))
        srcp = jnp.concatenate(
            [src.reshape(NW, e // NW),
             jnp.zeros((NW, ppt), jnp.int32)], axis=1).reshape(-1)
        dstp = jnp.concatenate(
            [dst.reshape(NW, e // NW),
             jnp.tile(spread, (NW, 1))], axis=1).reshape(-1)
    else:
        srcp = jnp.concatenate([src, jnp.zeros((padn,), jnp.int32)])
        dstp = jnp.concatenate([dst, jnp.full((padn,), dump, jnp.int32)])
    dst2d = dstp.reshape(NW * cpt, CH)
    zrows = jnp.zeros((40, DM), f32)

    grid = (g_blocks,)
    enc = pl.pallas_call(
        _tc_enc_body,
        grid=grid,
        in_specs=[
            pl.BlockSpec((RB, 1), lambda i: (i, 0)),
            pl.BlockSpec((RB, 8), lambda i: (i, 0)),
            _full((8, DM)), _full((NSP, DM)), _full((NB, DM)),
            _full((1, DM)), _full((2 * DM, DM)),
        ],
        out_specs=[_row_spec(), _row_spec(),
                   pl.BlockSpec((GPB, DM), lambda i: (i, 0))],
        out_shape=[jax.ShapeDtypeStruct((n, DM), f32),
                   jax.ShapeDtypeStruct((n, DM), f32),
                   jax.ShapeDtypeStruct((B, DM), f32)],
    )
    a1, bm1, g0 = enc(sem2d, geo8, wg2, tsem, tpos, b0, W1)

    cntp = _make_sc_cnt(ept, cnt_r)(dstp)
    cnt_t = cntp.reshape(NW, cnt_r).T
    sc_scatter = _make_sc_scatter(ept, cpt, acc_r)

    def mid(a, s0, s1, bl, wn):
        call = pl.pallas_call(
            _tc_mid_body,
            grid=grid,
            in_specs=[
                _row_spec(), _row_spec(), _row_spec(),
                pl.BlockSpec((RB, NW), lambda i: (i, 0)),
                _full((1, DM)), _full((2 * DM, DM)),
            ],
            out_specs=[_row_spec(), _row_spec(),
                       pl.BlockSpec((GPB, DM), lambda i: (i, 0))],
            out_shape=[jax.ShapeDtypeStruct((n, DM), f32),
                       jax.ShapeDtypeStruct((n, DM), f32),
                       jax.ShapeDtypeStruct((B, DM), f32)],
        )
        return call(a, s0, s1, cnt_t, bl[None, :], wn)

    s1a, s1b = sc_scatter(bm1, srcp, dst2d, zrows)
    a2, bm2, g1 = mid(a1, s1a, s1b, b1, W2)
    s2a, s2b = sc_scatter(bm2, srcp, dst2d, zrows)
    a3, bm3, g2 = mid(a2, s2a, s2b, b2, W3)
    s3a, s3b = sc_scatter(bm3, srcp, dst2d, zrows)

    last = pl.pallas_call(
        _tc_last_body,
        grid=grid,
        in_specs=[
            _row_spec(), _row_spec(), _row_spec(),
            pl.BlockSpec((RB, NW), lambda i: (i, 0)),
            _full((1, DM)),
        ],
        out_specs=[pl.BlockSpec((GPB, DM), lambda i: (i, 0))],
        out_shape=[jax.ShapeDtypeStruct((B, DM), f32)],
    )
    (g3,) = last(a3, s3a, s3b, cnt_t, b3[None, :])

    head = pl.pallas_call(
        _tc_head_body,
        grid=(1,),
        in_specs=[_full((B, DM))] * 4 + [
            _full((4 * DM, DM)), _full((1, DM)),
            _full((DM, DM)), _full((1, DM)),
            _full((DM, DM)), _full((1, DM)),
        ],
        out_specs=[_full((B, DM)), _full((B, DM))],
        out_shape=[jax.ShapeDtypeStruct((B, DM), f32),
                   jax.ShapeDtypeStruct((B, DM), f32)],
    )
    mu, log_var = head(g0, g1, g2, g3, Wagg, bagg[None, :], Wmu,
                       bmu[None, :], Wvar, bvar[None, :])
    return (mu, log_var)
